# baseline (speedup 1.0000x reference)
import functools

import numpy as np

import jax
import jax.numpy as jnp
from jax.experimental import pallas as pl
from jax.experimental.pallas import tpu as pltpu
from jax.sharding import Mesh, NamedSharding, PartitionSpec as P

try:
    from jax import shard_map as _shard_map
except ImportError:
    from jax.experimental.shard_map import shard_map as _shard_map


def _fused_block_kernel(x_ref, g_ref, beta_ref,
                        wqkv_ref, wo_ref, w1_ref, b1_ref, w2_ref, b2_ref,
                        o_ref, *, heads, inner, ffd):
    x = x_ref[...].astype(jnp.float32)
    mu = jnp.mean(x, axis=-1, keepdims=True)
    var = jnp.mean(jnp.square(x - mu), axis=-1, keepdims=True)
    xn = (x - mu) * jax.lax.rsqrt(var + 1e-5)
    xn = xn * g_ref[...].astype(jnp.float32) + beta_ref[...].astype(jnp.float32)
    xnb = xn.astype(jnp.bfloat16)

    qkv = jnp.dot(xnb, wqkv_ref[...],
                  preferred_element_type=jnp.float32).astype(jnp.bfloat16)
    q, k, v = (qkv[:, :inner], qkv[:, inner:2 * inner], qkv[:, 2 * inner:])

    h12 = jnp.dot(xnb, w1_ref[...],
                  preferred_element_type=jnp.float32) + b1_ref[...]
    hx, hg = h12[:, :ffd], h12[:, ffd:]
    sw = (hx * (hg * jax.nn.sigmoid(hg))).astype(jnp.bfloat16)
    acc = jnp.dot(sw, w2_ref[...],
                  preferred_element_type=jnp.float32) + b2_ref[...]

    dh = inner // heads
    for h in range(heads):
        sl = slice(h * dh, (h + 1) * dh)
        qh, kh, vh = q[:, sl], k[:, sl], v[:, sl]
        s = jax.lax.dot_general(qh, kh, (((1,), (1,)), ((), ())),
                                preferred_element_type=jnp.float32)
        m = jnp.max(s, axis=-1, keepdims=True)
        p = jnp.exp(s - m)
        inv_l = pl.reciprocal(jnp.sum(p, axis=-1, keepdims=True), approx=True)
        oh = jnp.dot(p.astype(jnp.bfloat16), vh,
                     preferred_element_type=jnp.float32) * inv_l
        acc = acc + jnp.dot(oh.astype(jnp.bfloat16), wo_ref[sl, :],
                            preferred_element_type=jnp.float32)

    o_ref[...] = acc.astype(o_ref.dtype)


def _run_block(x, gamma, beta, wqkv, wo, w1, b1, w2, b2, *, heads):
    B, N, D = x.shape
    inner = wo.shape[0]
    ffd = w2.shape[0]

    bmap = lambda b: (b, 0, 0)
    wmap = lambda b: (0, 0)
    full2 = lambda shape: pl.BlockSpec(shape, wmap)

    return pl.pallas_call(
        functools.partial(_fused_block_kernel, heads=heads, inner=inner,
                          ffd=ffd),
        out_shape=jax.ShapeDtypeStruct((B, N, D), x.dtype),
        grid_spec=pltpu.PrefetchScalarGridSpec(
            num_scalar_prefetch=0,
            grid=(B,),
            in_specs=[
                pl.BlockSpec((pl.Squeezed(), N, D), bmap),
                full2((1, D)), full2((1, D)),
                full2((D, 3 * inner)), full2((inner, D)),
                full2((D, 2 * ffd)), full2((1, 2 * ffd)),
                full2((ffd, D)), full2((1, D)),
            ],
            out_specs=pl.BlockSpec((pl.Squeezed(), N, D), bmap),
        ),
        compiler_params=pltpu.CompilerParams(
            dimension_semantics=("parallel",),
            vmem_limit_bytes=56 * 1024 * 1024),
    )(x, gamma, beta, wqkv, wo, w1, b1, w2, b2)


def kernel(x, gamma, beta, wq, wkv, wo, w1, b1, w2, b2):
    B = x.shape[0]
    heads = 4
    inner = wq.shape[1]
    scale = (inner // heads) ** -0.5
    bf = jnp.bfloat16

    wqkv = jnp.concatenate([wq * scale, wkv], axis=1).astype(bf)
    args = (x, gamma, beta, wqkv, wo.astype(bf), w1.astype(bf), b1,
            w2.astype(bf), b2)
    run = functools.partial(_run_block, heads=heads)

    devs = jax.devices()
    nd = len(devs) if len(devs) > 1 and B % len(devs) == 0 else 1
    if nd == 1:
        return run(*args)

    mesh = Mesh(np.array(devs[:nd]), ("d",))
    batched = NamedSharding(mesh, P("d"))
    repl = NamedSharding(mesh, P())
    sharded_args = (jax.device_put(args[0], batched),) + tuple(
        jax.device_put(a, repl) for a in args[1:])
    return _shard_map(run, mesh=mesh, in_specs=(P("d"),) + (P(),) * 8,
                      out_specs=P("d"), check_vma=False)(*sharded_args)


# speedup vs baseline: 4.6400x; 4.6400x over previous
import functools

import jax
import jax.numpy as jnp
from jax.experimental import pallas as pl
from jax.experimental.pallas import tpu as pltpu


def _fused_block_kernel(x_ref, g_ref, beta_ref,
                        wqkv_ref, wo_ref, w1_ref, b1_ref, w2_ref, b2_ref,
                        o_ref, *, heads, inner, ffd):
    x = x_ref[...].astype(jnp.float32)
    mu = jnp.mean(x, axis=-1, keepdims=True)
    var = jnp.mean(jnp.square(x - mu), axis=-1, keepdims=True)
    xn = (x - mu) * jax.lax.rsqrt(var + 1e-5)
    xn = xn * g_ref[...].astype(jnp.float32) + beta_ref[...].astype(jnp.float32)
    xnb = xn.astype(jnp.bfloat16)

    qkv = jnp.dot(xnb, wqkv_ref[...],
                  preferred_element_type=jnp.float32).astype(jnp.bfloat16)
    q, k, v = (qkv[:, :inner], qkv[:, inner:2 * inner], qkv[:, 2 * inner:])

    h12 = jnp.dot(xnb, w1_ref[...],
                  preferred_element_type=jnp.float32) + b1_ref[...]

    n = x.shape[0]
    dh = inner // heads
    nc = 1
    rc = n // nc
    for c in range(nc):
        r = slice(c * rc, (c + 1) * rc)
        hx, hg = h12[r, :ffd], h12[r, ffd:]
        sw = (hx * (hg * jax.nn.sigmoid(hg))).astype(jnp.bfloat16)
        acc = jnp.dot(sw, w2_ref[...],
                      preferred_element_type=jnp.float32) + b2_ref[...]
        ohs = []
        for h in range(heads):
            sl = slice(h * dh, (h + 1) * dh)
            qh, kh, vh = q[r, sl], k[:, sl], v[:, sl]
            s = jax.lax.dot_general(qh, kh, (((1,), (1,)), ((), ())),
                                    preferred_element_type=jnp.float32)
            m = jnp.max(s, axis=-1, keepdims=True)
            p = jnp.exp(s - m)
            inv_l = pl.reciprocal(jnp.sum(p, axis=-1, keepdims=True),
                                  approx=True)
            oh = jnp.dot(p.astype(jnp.bfloat16), vh,
                         preferred_element_type=jnp.float32) * inv_l
            ohs.append(oh.astype(jnp.bfloat16))
        o_all = jnp.concatenate(ohs, axis=1)
        acc = acc + jnp.dot(o_all, wo_ref[...],
                            preferred_element_type=jnp.float32)
        o_ref[r, :] = acc.astype(o_ref.dtype)


def _run_block(x, gamma, beta, wqkv, wo, w1, b1, w2, b2, *, heads):
    B, N, D = x.shape
    inner = wo.shape[0]
    ffd = w2.shape[0]

    bmap = lambda b: (b, 0, 0)
    wmap = lambda b: (0, 0)
    full2 = lambda shape: pl.BlockSpec(shape, wmap)

    return pl.pallas_call(
        functools.partial(_fused_block_kernel, heads=heads, inner=inner,
                          ffd=ffd),
        out_shape=jax.ShapeDtypeStruct((B, N, D), x.dtype),
        grid_spec=pltpu.PrefetchScalarGridSpec(
            num_scalar_prefetch=0,
            grid=(B,),
            in_specs=[
                pl.BlockSpec((pl.Squeezed(), N, D), bmap),
                full2((1, D)), full2((1, D)),
                full2((D, 3 * inner)), full2((inner, D)),
                full2((D, 2 * ffd)), full2((1, 2 * ffd)),
                full2((ffd, D)), full2((1, D)),
            ],
            out_specs=pl.BlockSpec((pl.Squeezed(), N, D), bmap),
        ),
        compiler_params=pltpu.CompilerParams(
            dimension_semantics=("parallel",),
            vmem_limit_bytes=56 * 1024 * 1024),
    )(x, gamma, beta, wqkv, wo, w1, b1, w2, b2)


def kernel(x, gamma, beta, wq, wkv, wo, w1, b1, w2, b2):
    B = x.shape[0]
    heads = 4
    inner = wq.shape[1]
    scale = (inner // heads) ** -0.5
    bf = jnp.bfloat16

    wqkv = jnp.concatenate([wq * scale, wkv], axis=1).astype(bf)
    args = (x, gamma, beta, wqkv, wo.astype(bf), w1.astype(bf), b1,
            w2.astype(bf), b2)
    run = functools.partial(_run_block, heads=heads)

    return run(*args)


# speedup vs baseline: 4.8708x; 1.0497x over previous
import functools

import jax
import jax.numpy as jnp
from jax.experimental import pallas as pl
from jax.experimental.pallas import tpu as pltpu


def _fused_block_kernel(x_ref, g_ref, beta_ref,
                        wqkv_ref, wo_ref, w1_ref, b1_ref, w2_ref, b2_ref,
                        o_ref, *, heads, inner, ffd, seq):
    x = x_ref[...].astype(jnp.float32)
    mu = jnp.mean(x, axis=-1, keepdims=True)
    var = jnp.mean(jnp.square(x - mu), axis=-1, keepdims=True)
    xn = (x - mu) * jax.lax.rsqrt(var + 1e-5)
    xn = xn * g_ref[...].astype(jnp.float32) + beta_ref[...].astype(jnp.float32)
    xnb = xn.astype(jnp.bfloat16)

    qkv = jnp.dot(xnb, wqkv_ref[...],
                  preferred_element_type=jnp.float32).astype(jnp.bfloat16)
    q, k, v = (qkv[:, :inner], qkv[:, inner:2 * inner], qkv[:, 2 * inner:])

    h12 = jnp.dot(xnb, w1_ref[...],
                  preferred_element_type=jnp.float32) + b1_ref[...]

    n = x.shape[0]
    dh = inner // heads
    for c in range(n // seq):
        r = slice(c * seq, (c + 1) * seq)
        ohs = []
        for h in range(heads):
            sl = slice(h * dh, (h + 1) * dh)
            qh, kh, vh = q[r, sl], k[r, sl], v[r, sl]
            s = jax.lax.dot_general(qh, kh, (((1,), (1,)), ((), ())),
                                    preferred_element_type=jnp.float32)
            m = jnp.max(s, axis=-1, keepdims=True)
            p = jnp.exp(s - m)
            inv_l = pl.reciprocal(jnp.sum(p, axis=-1, keepdims=True),
                                  approx=True)
            oh = jnp.dot(p.astype(jnp.bfloat16), vh,
                         preferred_element_type=jnp.float32) * inv_l
            ohs.append(oh.astype(jnp.bfloat16))
        hx, hg = h12[r, :ffd], h12[r, ffd:]
        sw = (hx * (hg * jax.nn.sigmoid(hg))).astype(jnp.bfloat16)
        acc = jnp.dot(sw, w2_ref[...],
                      preferred_element_type=jnp.float32) + b2_ref[...]
        o_all = jnp.concatenate(ohs, axis=1)
        acc = acc + jnp.dot(o_all, wo_ref[...],
                            preferred_element_type=jnp.float32)
        o_ref[r, :] = acc.astype(o_ref.dtype)


def _run_block(x, gamma, beta, wqkv, wo, w1, b1, w2, b2, *, heads, seq):
    B, N, D = x.shape
    inner = wo.shape[0]
    ffd = w2.shape[0]

    bmap = lambda b: (b, 0, 0)
    wmap = lambda b: (0, 0)
    full2 = lambda shape: pl.BlockSpec(shape, wmap)

    return pl.pallas_call(
        functools.partial(_fused_block_kernel, heads=heads, inner=inner,
                          ffd=ffd, seq=seq),
        out_shape=jax.ShapeDtypeStruct((B, N, D), x.dtype),
        grid_spec=pltpu.PrefetchScalarGridSpec(
            num_scalar_prefetch=0,
            grid=(B,),
            in_specs=[
                pl.BlockSpec((pl.Squeezed(), N, D), bmap),
                full2((1, D)), full2((1, D)),
                full2((D, 3 * inner)), full2((inner, D)),
                full2((D, 2 * ffd)), full2((1, 2 * ffd)),
                full2((ffd, D)), full2((1, D)),
            ],
            out_specs=pl.BlockSpec((pl.Squeezed(), N, D), bmap),
        ),
        compiler_params=pltpu.CompilerParams(
            dimension_semantics=("parallel",),
            vmem_limit_bytes=56 * 1024 * 1024),
    )(x, gamma, beta, wqkv, wo, w1, b1, w2, b2)


def kernel(x, gamma, beta, wq, wkv, wo, w1, b1, w2, b2):
    B, N, D = x.shape
    heads = 4
    inner = wq.shape[1]
    scale = (inner // heads) ** -0.5
    bf = jnp.bfloat16

    wqkv = jnp.concatenate([wq * scale, wkv], axis=1).astype(bf)
    wargs = (gamma, beta, wqkv, wo.astype(bf), w1.astype(bf), b1,
             w2.astype(bf), b2)

    pack = 2 if B % 2 == 0 else 1
    xr = x.reshape(B // pack, pack * N, D)
    out = _run_block(xr, *wargs, heads=heads, seq=N)
    return out.reshape(B, N, D)


# speedup vs baseline: 4.9168x; 1.0095x over previous
import functools

import jax
import jax.numpy as jnp
from jax.experimental import pallas as pl
from jax.experimental.pallas import tpu as pltpu


def _fused_block_kernel(x_ref, g_ref, beta_ref,
                        wqkv_ref, wo_ref, w1_ref, b1_ref, w2_ref, b2_ref,
                        o_ref, *, heads, inner, ffd, seq):
    x = x_ref[...].astype(jnp.float32)
    mu = jnp.mean(x, axis=-1, keepdims=True)
    var = jnp.mean(jnp.square(x - mu), axis=-1, keepdims=True)
    xn = (x - mu) * jax.lax.rsqrt(var + 1e-5)
    xn = xn * g_ref[...].astype(jnp.float32) + beta_ref[...].astype(jnp.float32)
    xnb = xn.astype(jnp.bfloat16)

    qkv = jnp.dot(xnb, wqkv_ref[...],
                  preferred_element_type=jnp.float32).astype(jnp.bfloat16)
    q, k, v = (qkv[:, :inner], qkv[:, inner:2 * inner], qkv[:, 2 * inner:])

    n = x.shape[0]
    dh = inner // heads
    for c in range(n // seq):
        r = slice(c * seq, (c + 1) * seq)
        h12 = jnp.dot(xnb[r], w1_ref[...],
                      preferred_element_type=jnp.float32) + b1_ref[...]
        ohs = []
        for h in range(heads):
            sl = slice(h * dh, (h + 1) * dh)
            qh, kh, vh = q[r, sl], k[r, sl], v[r, sl]
            s = jax.lax.dot_general(qh, kh, (((1,), (1,)), ((), ())),
                                    preferred_element_type=jnp.float32)
            m = jnp.max(s, axis=-1, keepdims=True)
            p = jnp.exp(s - m)
            inv_l = pl.reciprocal(jnp.sum(p, axis=-1, keepdims=True),
                                  approx=True)
            oh = jnp.dot(p.astype(jnp.bfloat16), vh,
                         preferred_element_type=jnp.float32) * inv_l
            ohs.append(oh.astype(jnp.bfloat16))
        hx, hg = h12[:, :ffd], h12[:, ffd:]
        sw = (hx * (hg * jax.nn.sigmoid(hg))).astype(jnp.bfloat16)
        acc = jnp.dot(sw, w2_ref[...],
                      preferred_element_type=jnp.float32) + b2_ref[...]
        o_all = jnp.concatenate(ohs, axis=1)
        acc = acc + jnp.dot(o_all, wo_ref[...],
                            preferred_element_type=jnp.float32)
        o_ref[r, :] = acc.astype(o_ref.dtype)


def _run_block(x, gamma, beta, wqkv, wo, w1, b1, w2, b2, *, heads, seq):
    B, N, D = x.shape
    inner = wo.shape[0]
    ffd = w2.shape[0]

    bmap = lambda b: (b, 0, 0)
    wmap = lambda b: (0, 0)
    full2 = lambda shape: pl.BlockSpec(shape, wmap)

    return pl.pallas_call(
        functools.partial(_fused_block_kernel, heads=heads, inner=inner,
                          ffd=ffd, seq=seq),
        out_shape=jax.ShapeDtypeStruct((B, N, D), x.dtype),
        grid_spec=pltpu.PrefetchScalarGridSpec(
            num_scalar_prefetch=0,
            grid=(B,),
            in_specs=[
                pl.BlockSpec((pl.Squeezed(), N, D), bmap),
                full2((1, D)), full2((1, D)),
                full2((D, 3 * inner)), full2((inner, D)),
                full2((D, 2 * ffd)), full2((1, 2 * ffd)),
                full2((ffd, D)), full2((1, D)),
            ],
            out_specs=pl.BlockSpec((pl.Squeezed(), N, D), bmap),
        ),
        compiler_params=pltpu.CompilerParams(
            dimension_semantics=("parallel",),
            vmem_limit_bytes=56 * 1024 * 1024),
    )(x, gamma, beta, wqkv, wo, w1, b1, w2, b2)


def kernel(x, gamma, beta, wq, wkv, wo, w1, b1, w2, b2):
    B, N, D = x.shape
    heads = 4
    inner = wq.shape[1]
    scale = (inner // heads) ** -0.5
    bf = jnp.bfloat16

    wqkv = jnp.concatenate([wq * scale, wkv], axis=1).astype(bf)
    wargs = (gamma, beta, wqkv, wo.astype(bf), w1.astype(bf), b1,
             w2.astype(bf), b2)

    pack = 2 if B % 2 == 0 else 1
    xr = x.reshape(B // pack, pack * N, D)
    out = _run_block(xr, *wargs, heads=heads, seq=N)
    return out.reshape(B, N, D)


# speedup vs baseline: 4.9530x; 1.0074x over previous
import functools

import jax
import jax.numpy as jnp
from jax.experimental import pallas as pl
from jax.experimental.pallas import tpu as pltpu


def _fused_block_kernel(x_ref, g_ref, beta_ref,
                        wqkv_ref, wo_ref, w1_ref, b1_ref, w2_ref, b2_ref,
                        o_ref, *, heads, inner, ffd, seq):
    n = x_ref.shape[0]
    dh = inner // heads
    for c in range(n // seq):
        r = slice(c * seq, (c + 1) * seq)
        x = x_ref[r, :].astype(jnp.float32)
        mu = jnp.mean(x, axis=-1, keepdims=True)
        var = jnp.mean(jnp.square(x - mu), axis=-1, keepdims=True)
        xn = (x - mu) * jax.lax.rsqrt(var + 1e-5)
        xn = (xn * g_ref[...].astype(jnp.float32)
              + beta_ref[...].astype(jnp.float32))
        xnb = xn.astype(jnp.bfloat16)

        qkv = jnp.dot(xnb, wqkv_ref[...],
                      preferred_element_type=jnp.float32).astype(jnp.bfloat16)
        q, k, v = (qkv[:, :inner], qkv[:, inner:2 * inner], qkv[:, 2 * inner:])

        h12 = jnp.dot(xnb, w1_ref[...],
                      preferred_element_type=jnp.float32) + b1_ref[...]
        ohs = []
        for h in range(heads):
            sl = slice(h * dh, (h + 1) * dh)
            qh, kh, vh = q[:, sl], k[:, sl], v[:, sl]
            s = jax.lax.dot_general(qh, kh, (((1,), (1,)), ((), ())),
                                    preferred_element_type=jnp.float32)
            m = jnp.max(s, axis=-1, keepdims=True)
            p = jnp.exp(s - m)
            inv_l = pl.reciprocal(jnp.sum(p, axis=-1, keepdims=True),
                                  approx=True)
            oh = jnp.dot(p.astype(jnp.bfloat16), vh,
                         preferred_element_type=jnp.float32) * inv_l
            ohs.append(oh.astype(jnp.bfloat16))
        hx, hg = h12[:, :ffd], h12[:, ffd:]
        sw = (hx * (hg * jax.nn.sigmoid(hg))).astype(jnp.bfloat16)
        acc = jnp.dot(sw, w2_ref[...],
                      preferred_element_type=jnp.float32) + b2_ref[...]
        o_all = jnp.concatenate(ohs, axis=1)
        acc = acc + jnp.dot(o_all, wo_ref[...],
                            preferred_element_type=jnp.float32)
        o_ref[r, :] = acc.astype(o_ref.dtype)


def _run_block(x, gamma, beta, wqkv, wo, w1, b1, w2, b2, *, heads, seq):
    B, N, D = x.shape
    inner = wo.shape[0]
    ffd = w2.shape[0]

    bmap = lambda b: (b, 0, 0)
    wmap = lambda b: (0, 0)
    full2 = lambda shape: pl.BlockSpec(shape, wmap)

    return pl.pallas_call(
        functools.partial(_fused_block_kernel, heads=heads, inner=inner,
                          ffd=ffd, seq=seq),
        out_shape=jax.ShapeDtypeStruct((B, N, D), x.dtype),
        grid_spec=pltpu.PrefetchScalarGridSpec(
            num_scalar_prefetch=0,
            grid=(B,),
            in_specs=[
                pl.BlockSpec((pl.Squeezed(), N, D), bmap),
                full2((1, D)), full2((1, D)),
                full2((D, 3 * inner)), full2((inner, D)),
                full2((D, 2 * ffd)), full2((1, 2 * ffd)),
                full2((ffd, D)), full2((1, D)),
            ],
            out_specs=pl.BlockSpec((pl.Squeezed(), N, D), bmap),
        ),
        compiler_params=pltpu.CompilerParams(
            dimension_semantics=("parallel",),
            vmem_limit_bytes=56 * 1024 * 1024),
    )(x, gamma, beta, wqkv, wo, w1, b1, w2, b2)


def kernel(x, gamma, beta, wq, wkv, wo, w1, b1, w2, b2):
    B, N, D = x.shape
    heads = 4
    inner = wq.shape[1]
    scale = (inner // heads) ** -0.5
    bf = jnp.bfloat16

    wqkv = jnp.concatenate([wq * scale, wkv], axis=1).astype(bf)
    wargs = (gamma, beta, wqkv, wo.astype(bf), w1.astype(bf), b1,
             w2.astype(bf), b2)

    pack = 2 if B % 2 == 0 else 1
    xr = x.reshape(B // pack, pack * N, D)
    out = _run_block(xr, *wargs, heads=heads, seq=N)
    return out.reshape(B, N, D)


# speedup vs baseline: 4.9964x; 1.0088x over previous
import functools

import jax
import jax.numpy as jnp
from jax.experimental import pallas as pl
from jax.experimental.pallas import tpu as pltpu


def _fused_block_kernel(x_ref, g_ref, beta_ref,
                        wqkv_ref, wo_ref, w1_ref, b1_ref, w2_ref, b2_ref,
                        o_ref, *, heads, inner, ffd, seq):
    n = x_ref.shape[0]
    dh = inner // heads
    for c in range(n // seq):
        r = slice(c * seq, (c + 1) * seq)
        x = x_ref[r, :].astype(jnp.float32)
        mu = jnp.mean(x, axis=-1, keepdims=True)
        ex2 = jnp.mean(jnp.square(x), axis=-1, keepdims=True)
        var = ex2 - jnp.square(mu)
        xn = (x - mu) * jax.lax.rsqrt(var + 1e-5)
        xn = (xn * g_ref[...].astype(jnp.float32)
              + beta_ref[...].astype(jnp.float32))
        xnb = xn.astype(jnp.bfloat16)

        qkv = jnp.dot(xnb, wqkv_ref[...],
                      preferred_element_type=jnp.float32).astype(jnp.bfloat16)
        q, k, v = (qkv[:, :inner], qkv[:, inner:2 * inner], qkv[:, 2 * inner:])

        h12 = jnp.dot(xnb, w1_ref[...],
                      preferred_element_type=jnp.float32) + b1_ref[...]
        ohs = []
        for h in range(heads):
            sl = slice(h * dh, (h + 1) * dh)
            qh, kh, vh = q[:, sl], k[:, sl], v[:, sl]
            s = jax.lax.dot_general(qh, kh, (((1,), (1,)), ((), ())),
                                    preferred_element_type=jnp.float32)
            m = jnp.max(s, axis=-1, keepdims=True)
            p = jnp.exp2(s - m)
            inv_l = pl.reciprocal(jnp.sum(p, axis=-1, keepdims=True),
                                  approx=True)
            oh = jnp.dot(p.astype(jnp.bfloat16), vh,
                         preferred_element_type=jnp.float32) * inv_l
            ohs.append(oh.astype(jnp.bfloat16))
        hx, hg = h12[:, :ffd], h12[:, ffd:]
        sw = (hx * (hg * jax.nn.sigmoid(hg))).astype(jnp.bfloat16)
        acc = jnp.dot(sw, w2_ref[...],
                      preferred_element_type=jnp.float32) + b2_ref[...]
        o_all = jnp.concatenate(ohs, axis=1)
        acc = acc + jnp.dot(o_all, wo_ref[...],
                            preferred_element_type=jnp.float32)
        o_ref[r, :] = acc.astype(o_ref.dtype)


def _run_block(x, gamma, beta, wqkv, wo, w1, b1, w2, b2, *, heads, seq):
    B, N, D = x.shape
    inner = wo.shape[0]
    ffd = w2.shape[0]

    bmap = lambda b: (b, 0, 0)
    wmap = lambda b: (0, 0)
    full2 = lambda shape: pl.BlockSpec(shape, wmap)

    return pl.pallas_call(
        functools.partial(_fused_block_kernel, heads=heads, inner=inner,
                          ffd=ffd, seq=seq),
        out_shape=jax.ShapeDtypeStruct((B, N, D), x.dtype),
        grid_spec=pltpu.PrefetchScalarGridSpec(
            num_scalar_prefetch=0,
            grid=(B,),
            in_specs=[
                pl.BlockSpec((pl.Squeezed(), N, D), bmap),
                full2((1, D)), full2((1, D)),
                full2((D, 3 * inner)), full2((inner, D)),
                full2((D, 2 * ffd)), full2((1, 2 * ffd)),
                full2((ffd, D)), full2((1, D)),
            ],
            out_specs=pl.BlockSpec((pl.Squeezed(), N, D), bmap),
        ),
        compiler_params=pltpu.CompilerParams(
            dimension_semantics=("parallel",),
            vmem_limit_bytes=56 * 1024 * 1024),
    )(x, gamma, beta, wqkv, wo, w1, b1, w2, b2)


def kernel(x, gamma, beta, wq, wkv, wo, w1, b1, w2, b2):
    B, N, D = x.shape
    heads = 4
    inner = wq.shape[1]
    scale = (inner // heads) ** -0.5
    bf = jnp.bfloat16

    wqkv = jnp.concatenate([wq * (scale * 1.4426950408889634), wkv],
                           axis=1).astype(bf)
    wargs = (gamma, beta, wqkv, wo.astype(bf), w1.astype(bf), b1,
             w2.astype(bf), b2)

    pack = 2 if B % 2 == 0 else 1
    xr = x.reshape(B // pack, pack * N, D)
    out = _run_block(xr, *wargs, heads=heads, seq=N)
    return out.reshape(B, N, D)
